# baseline (speedup 1.0000x reference)
import functools

import jax
import jax.numpy as jnp
from jax.experimental import pallas as pl

_INPUT_SIZE = 128
_T = 16
_V = 64


def _decode_block(x_ref, a_ref, seq_ref, score_ref, *, bn):
    lt = jax.lax.dot_general(
        a_ref[...], x_ref[...],
        dimension_numbers=(((1,), (1,)), ((), ())),
        preferred_element_type=jnp.float32,
    )
    l3 = lt.reshape(_T, _V, bn)
    maxv = jnp.max(l3, axis=1)
    hit = l3 == maxv[:, None, :]
    rev = jax.lax.broadcasted_iota(jnp.int32, (_T, _V, bn), 1)
    rev = (jnp.int32(_V - 1) - rev).astype(jnp.float32)
    idxf = jnp.max(jnp.where(hit, rev, jnp.float32(-1)), axis=1)
    idx = (jnp.float32(_V - 1) - idxf).astype(jnp.int32)
    seq_ref[...] = idx.T
    score_ref[...] = jnp.sum(maxv, axis=0, keepdims=True)


def kernel(observation, W):
    batch, d = observation.shape
    ipo = d // _INPUT_SIZE
    n = batch * ipo
    x = observation.reshape(n, _INPUT_SIZE)
    a = W.reshape(_INPUT_SIZE, _T * _V).T

    bn = 2048
    grid = (n // bn,)
    seqs, scores_t = pl.pallas_call(
        functools.partial(_decode_block, bn=bn),
        grid=grid,
        in_specs=[
            pl.BlockSpec((bn, _INPUT_SIZE), lambda i: (i, 0)),
            pl.BlockSpec((_T * _V, _INPUT_SIZE), lambda i: (0, 0)),
        ],
        out_specs=[
            pl.BlockSpec((bn, _T), lambda i: (i, 0)),
            pl.BlockSpec((1, bn), lambda i: (0, i)),
        ],
        out_shape=[
            jax.ShapeDtypeStruct((n, _T), jnp.int32),
            jax.ShapeDtypeStruct((1, n), jnp.float32),
        ],
    )(x, a)

    seq_supp_batch = seqs.reshape(batch, ipo, _T)
    score_batch = scores_t.reshape(batch, ipo)
    length_supp_batch = jnp.full((batch, ipo), _T, dtype=jnp.int32)
    return seq_supp_batch, length_supp_batch, score_batch


# speedup vs baseline: 1.9865x; 1.9865x over previous
import functools

import jax
import jax.numpy as jnp
from jax.experimental import pallas as pl

_INPUT_SIZE = 128
_T = 16
_V = 64


def _decode_block(x_ref, a_ref, seq_ref, score_ref, *, b, jpb):
    for j in range(jpb):
        xj = x_ref[:, j * _INPUT_SIZE:(j + 1) * _INPUT_SIZE]
        lt = jax.lax.dot_general(
            a_ref[...], xj,
            dimension_numbers=(((1,), (1,)), ((), ())),
            preferred_element_type=jnp.float32,
        )
        l3 = lt.reshape(_T, _V, b)
        maxv = jnp.max(l3, axis=1)
        hit = l3 == maxv[:, None, :]
        rev = jax.lax.broadcasted_iota(jnp.int32, (_T, _V, b), 1)
        rev = (jnp.int32(_V - 1) - rev).astype(jnp.float32)
        idxf = jnp.max(jnp.where(hit, rev, jnp.float32(-1)), axis=1)
        idx = (jnp.float32(_V - 1) - idxf).astype(jnp.int32)
        seq_ref[j] = idx
        score_ref[j, 0, :] = jnp.sum(maxv, axis=0)


def kernel(observation, W):
    batch, d = observation.shape
    ipo = d // _INPUT_SIZE
    a = W.reshape(_INPUT_SIZE, _T * _V).T

    jpb = 4
    grid = (ipo // jpb,)
    seqs_t, scores_t = pl.pallas_call(
        functools.partial(_decode_block, b=batch, jpb=jpb),
        grid=grid,
        in_specs=[
            pl.BlockSpec((batch, jpb * _INPUT_SIZE), lambda i: (0, i)),
            pl.BlockSpec((_T * _V, _INPUT_SIZE), lambda i: (0, 0)),
        ],
        out_specs=[
            pl.BlockSpec((jpb, _T, batch), lambda i: (i, 0, 0)),
            pl.BlockSpec((jpb, 1, batch), lambda i: (i, 0, 0)),
        ],
        out_shape=[
            jax.ShapeDtypeStruct((ipo, _T, batch), jnp.int32),
            jax.ShapeDtypeStruct((ipo, 1, batch), jnp.float32),
        ],
    )(observation, a)

    seq_supp_batch = jnp.transpose(seqs_t, (2, 0, 1))
    score_batch = scores_t.reshape(ipo, batch).T
    length_supp_batch = jnp.full((batch, ipo), _T, dtype=jnp.int32)
    return seq_supp_batch, length_supp_batch, score_batch
